# baseline (speedup 1.0000x reference)
import jax
import jax.numpy as jnp
from jax.experimental import pallas as pl

_K = 64
_BM = 2048
_NEG = -1e30


def _accum_stats(y, m, s1_ref, s2_ref):
    @pl.when(pl.program_id(0) == 0)
    def _init():
        s1_ref[...] = jnp.zeros(s1_ref.shape, jnp.float32)
        s2_ref[...] = jnp.zeros(s2_ref.shape, jnp.float32)

    ym = y * m
    s1_ref[0:1, :] = s1_ref[0:1, :] + jnp.sum(ym, axis=0, keepdims=True)
    s2_ref[0:1, :] = s2_ref[0:1, :] + jnp.sum(ym * y, axis=0, keepdims=True)


def _lin_stats_kernel(h_ref, m_ref, w_ref, b_ref, y_ref, s1_ref, s2_ref):
    h = h_ref[...]
    y = jnp.dot(h, w_ref[...], preferred_element_type=jnp.float32) + b_ref[...]
    y_ref[...] = y
    _accum_stats(y, m_ref[...], s1_ref, s2_ref)


def _bn_lin_stats_kernel(h_ref, m_ref, sc_ref, sh_ref, w_ref, b_ref,
                         y_ref, s1_ref, s2_ref):
    h = jax.nn.relu(h_ref[...] * sc_ref[...] + sh_ref[...])
    y = jnp.dot(h, w_ref[...], preferred_element_type=jnp.float32) + b_ref[...]
    y_ref[...] = y
    _accum_stats(y, m_ref[...], s1_ref, s2_ref)


def _bn_lin_max_kernel(h_ref, m_ref, sc_ref, sh_ref, w_ref, b_ref, o_ref):
    h = jax.nn.relu(h_ref[...] * sc_ref[...] + sh_ref[...])
    y = jnp.dot(h, w_ref[...], preferred_element_type=jnp.float32) + b_ref[...]
    y = jnp.where(m_ref[...] > 0.5, y, _NEG)
    fout = y.shape[-1]
    y = y.reshape(y.shape[0] // _K, _K, fout)
    o_ref[...] = jnp.max(y, axis=1)


def _bn_lin_segmax_kernel(h_ref, bt_ref, sc_ref, sh_ref, w_ref, b_ref, o_ref):
    @pl.when(pl.program_id(0) == 0)
    def _init():
        o_ref[...] = jnp.full(o_ref.shape, _NEG, jnp.float32)

    h = jax.nn.relu(h_ref[...] * sc_ref[...] + sh_ref[...])
    y = jnp.dot(h, w_ref[...], preferred_element_type=jnp.float32) + b_ref[...]
    bt = bt_ref[...]
    for c in range(o_ref.shape[0]):
        yc = jnp.where(bt == c, y, _NEG)
        mx = jnp.max(yc, axis=0, keepdims=True)
        o_ref[c:c + 1, :] = jnp.maximum(o_ref[c:c + 1, :], mx)


def _linear_stats(h, m, w, b, first, sc=None, sh=None):
    mrows, fin = h.shape
    fout = w.shape[1]
    grid = mrows // _BM
    out_shape = [jax.ShapeDtypeStruct((mrows, fout), jnp.float32),
                 jax.ShapeDtypeStruct((8, fout), jnp.float32),
                 jax.ShapeDtypeStruct((8, fout), jnp.float32)]
    out_specs = [pl.BlockSpec((_BM, fout), lambda i: (i, 0)),
                 pl.BlockSpec((8, fout), lambda i: (0, 0)),
                 pl.BlockSpec((8, fout), lambda i: (0, 0))]
    base_specs = [pl.BlockSpec((_BM, fin), lambda i: (i, 0)),
                  pl.BlockSpec((_BM, 1), lambda i: (i, 0))]
    tail_specs = [pl.BlockSpec((fin, fout), lambda i: (0, 0)),
                  pl.BlockSpec((1, fout), lambda i: (0, 0))]
    if first:
        y, s1, s2 = pl.pallas_call(
            _lin_stats_kernel, grid=(grid,),
            in_specs=base_specs + tail_specs,
            out_specs=out_specs, out_shape=out_shape,
        )(h, m, w, b)
    else:
        coef_specs = [pl.BlockSpec((1, fin), lambda i: (0, 0)),
                      pl.BlockSpec((1, fin), lambda i: (0, 0))]
        y, s1, s2 = pl.pallas_call(
            _bn_lin_stats_kernel, grid=(grid,),
            in_specs=base_specs + coef_specs + tail_specs,
            out_specs=out_specs, out_shape=out_shape,
        )(h, m, sc, sh, w, b)
    return y, jnp.sum(s1, axis=0), jnp.sum(s2, axis=0)


def _bn_linear_max(h, m, sc, sh, w, b):
    mrows, fin = h.shape
    fout = w.shape[1]
    grid = mrows // _BM
    npts = mrows // _K
    return pl.pallas_call(
        _bn_lin_max_kernel, grid=(grid,),
        in_specs=[pl.BlockSpec((_BM, fin), lambda i: (i, 0)),
                  pl.BlockSpec((_BM, 1), lambda i: (i, 0)),
                  pl.BlockSpec((1, fin), lambda i: (0, 0)),
                  pl.BlockSpec((1, fin), lambda i: (0, 0)),
                  pl.BlockSpec((fin, fout), lambda i: (0, 0)),
                  pl.BlockSpec((1, fout), lambda i: (0, 0))],
        out_specs=pl.BlockSpec((_BM // _K, fout), lambda i: (i, 0)),
        out_shape=jax.ShapeDtypeStruct((npts, fout), jnp.float32),
    )(h, m, sc, sh, w, b)


def _bn_linear_segmax(h, bt, sc, sh, w, b, num_seg):
    mrows, fin = h.shape
    fout = w.shape[1]
    grid = mrows // _BM
    return pl.pallas_call(
        _bn_lin_segmax_kernel, grid=(grid,),
        in_specs=[pl.BlockSpec((_BM, fin), lambda i: (i, 0)),
                  pl.BlockSpec((_BM, 1), lambda i: (i, 0)),
                  pl.BlockSpec((1, fin), lambda i: (0, 0)),
                  pl.BlockSpec((1, fin), lambda i: (0, 0)),
                  pl.BlockSpec((fin, fout), lambda i: (0, 0)),
                  pl.BlockSpec((1, fout), lambda i: (0, 0))],
        out_specs=pl.BlockSpec((num_seg, fout), lambda i: (0, 0)),
        out_shape=jax.ShapeDtypeStruct((num_seg, fout), jnp.float32),
    )(h, bt, sc, sh, w, b)


def _bn_coeffs(s1, s2, cnt, g, be):
    mean = s1 / cnt
    var = jnp.maximum(s2 / cnt - mean * mean, 0.0)
    scale = g / jnp.sqrt(var + 1e-5)
    shift = be - mean * scale
    return scale[None, :], shift[None, :]


def _sa(x, pos, nbr, mask, layers):
    npts, kk = nbr.shape
    xj = x[nbr]
    rel = pos[nbr] - pos[:, None, :]
    h0 = jnp.concatenate([xj, rel], axis=-1).reshape(npts * kk, -1)
    m = mask.reshape(-1, 1).astype(jnp.float32)
    cnt = jnp.maximum(jnp.sum(m), 1.0)
    l1, l2, l3 = layers
    y1, s1, s2 = _linear_stats(h0, m, l1['W'], l1['b'][None, :], True)
    sc1, sh1 = _bn_coeffs(s1, s2, cnt, l1['g'], l1['be'])
    y2, t1, t2 = _linear_stats(y1, m, l2['W'], l2['b'][None, :], False, sc1, sh1)
    sc2, sh2 = _bn_coeffs(t1, t2, cnt, l2['g'], l2['be'])
    return _bn_linear_max(y2, m, sc2, sh2, l3['W'], l3['b'][None, :])


def kernel(x, pos, batch, params):
    npts = pos.shape[0]
    p = jax.lax.stop_gradient(pos)
    d2 = jnp.sum((p[:, None, :] - p[None, :, :]) ** 2, axis=-1)
    same = batch[:, None] == batch[None, :]
    eye = jnp.eye(npts, dtype=jnp.float32)

    def neighbors(r):
        valid = jnp.logical_and(same, d2 <= r * r)
        score = valid.astype(jnp.float32) + eye
        vals, nbr = jax.lax.top_k(score, _K)
        return nbr, vals > 0.5

    nbr1, m1 = neighbors(0.1)
    x1 = _sa(x, pos, nbr1, m1, params['mlp1'])
    nbr2, m2 = neighbors(0.2)
    x2 = _sa(x1, pos, nbr2, m2, params['mlp2'])

    num_seg = 4
    h = jnp.concatenate([x2, pos], axis=1)
    m = jnp.ones((npts, 1), jnp.float32)
    cnt = jnp.asarray(npts, jnp.float32)
    l1, l2, l3 = params['mlp3']
    y1, s1, s2 = _linear_stats(h, m, l1['W'], l1['b'][None, :], True)
    sc1, sh1 = _bn_coeffs(s1, s2, cnt, l1['g'], l1['be'])
    y2, t1, t2 = _linear_stats(y1, m, l2['W'], l2['b'][None, :], False, sc1, sh1)
    sc2, sh2 = _bn_coeffs(t1, t2, cnt, l2['g'], l2['be'])
    bt = batch.astype(jnp.int32)[:, None]
    x3 = _bn_linear_segmax(y2, bt, sc2, sh2, l3['W'], l3['b'][None, :], num_seg)

    pos3 = jnp.zeros((num_seg, 3), dtype=pos.dtype)
    batch3 = jnp.arange(num_seg, dtype=jnp.int32)
    return (x, pos, batch, x1, pos, batch, x2, pos, batch, x3, pos3, batch3)
